# Initial kernel scaffold; baseline (speedup 1.0000x reference)
import jax
import jax.numpy as jnp
from jax.experimental import pallas as pl


def kernel(x, edge_index, batch, params):
    raise NotImplementedError("write your pallas kernel here")


# speedup vs baseline: 5.8182x; 5.8182x over previous
import functools

import jax
import jax.numpy as jnp
from jax import lax
from jax.experimental import pallas as pl
from jax.experimental.pallas import tpu as pltpu
from jax.experimental.pallas import tpu_sc as plsc

N = 10000
E = 160000
NG = 16
ROWS = 400
GRID = N // ROWS
NPAD = 12800
ACC = 10112
SUB = 16
BLK = 128
EPT = E // SUB
NBLK = 79
EPT_PAD = NBLK * BLK
EPS = 1e-7
BN_EPS = 1e-5
NEG = -1e38


@functools.cache
def _sc_edge(C):
  cpc = C // 2
  rps = ACC // SUB
  mesh = plsc.VectorSubcoreMesh(core_axis_name="c", subcore_axis_name="s")

  @functools.partial(
      pl.kernel,
      out_type=jax.ShapeDtypeStruct((C * NPAD, 128), jnp.float32),
      mesh=mesh,
      scratch_types=[
          pltpu.VMEM_SHARED((ACC, 128), jnp.float32),
          pltpu.VMEM((NBLK, 2, BLK), jnp.int32),
          pltpu.VMEM((BLK, 128), jnp.float32),
          pltpu.SemaphoreType.DMA,
      ],
  )
  def k(table, pack, zeros, out, acc, packv, bufa, sema):
    core = lax.axis_index("c")
    sub = lax.axis_index("s")
    row0 = sub * rps
    for j in range(cpc):
      chunk = core * cpc + j
      pltpu.sync_copy(pack.at[chunk * SUB + sub], packv)
      pltpu.sync_copy(zeros, acc.at[pl.ds(row0, rps)])
      plsc.subcore_barrier()

      pltpu.async_copy(table.at[packv.at[0, 0]], bufa, sema)

      @pl.loop(0, NBLK)
      def _(b):
        pltpu.make_async_copy(table.at[packv.at[0, 0]], bufa, sema).wait()
        pltpu.sync_copy(bufa, acc.at[packv.at[b, 1]], add=True)

        @pl.when(b + 1 < NBLK)
        def _():
          pltpu.async_copy(table.at[packv.at[b + 1, 0]], bufa, sema)

      plsc.subcore_barrier()
      pltpu.sync_copy(acc.at[pl.ds(row0, rps)],
                      out.at[pl.ds(chunk * NPAD + row0, rps)])

  return k


@functools.cache
def _prep(F):
  C = 2 * F // 128
  half = F // 128

  def body(t_ref, x_ref, out_ref):
    x = x_ref[...]
    m = jnp.maximum(x, 0.0) + EPS
    w = jnp.exp(m * t_ref[0])
    p = m * w
    for c in range(half):
      out_ref[c] = w[:, c * 128:(c + 1) * 128]
      out_ref[half + c] = p[:, c * 128:(c + 1) * 128]

  return pl.pallas_call(
      body,
      grid=(GRID,),
      in_specs=[
          pl.BlockSpec(memory_space=pltpu.SMEM),
          pl.BlockSpec((ROWS, F), lambda i: (i, 0)),
      ],
      out_specs=pl.BlockSpec((C, ROWS, 128), lambda i: (0, i, 0)),
      out_shape=jax.ShapeDtypeStruct((C, N, 128), jnp.float32),
  )


@functools.cache
def _combine(F, FH):
  C = 2 * F // 128
  half = F // 128
  blocks_per_chunk = NPAD // ROWS

  def body(s_ref, x_ref, *refs):
    i = pl.program_id(0)
    chunks = refs[:C]
    w1_ref = refs[C]
    z_ref = refs[C + 1]
    stats_ref = refs[C + 2]
    den = jnp.concatenate([chunks[c][...] for c in range(half)], axis=1)
    num = jnp.concatenate([chunks[half + c][...] for c in range(half)],
                          axis=1)
    x = x_ref[...]
    aggr = num / (den + 1e-16)
    nrm = jnp.sqrt(jnp.sum(aggr * aggr, axis=1, keepdims=True))
    aggr_n = aggr / jnp.maximum(nrm, 1e-12)
    xn = jnp.sqrt(jnp.sum(x * x, axis=1, keepdims=True))
    h = x + aggr_n * xn * s_ref[0]
    z = jnp.dot(h, w1_ref[...], preferred_element_type=jnp.float32)
    z_ref[...] = z
    ps = jnp.sum(z, axis=0, keepdims=True)
    pss = jnp.sum(z * z, axis=0, keepdims=True)
    upd = jnp.concatenate([ps, pss, jnp.zeros((6, FH), jnp.float32)], axis=0)

    @pl.when(i == 0)
    def _():
      stats_ref[...] = upd

    @pl.when(i > 0)
    def _():
      stats_ref[...] = stats_ref[...] + upd

  in_specs = [
      pl.BlockSpec(memory_space=pltpu.SMEM),
      pl.BlockSpec((ROWS, F), lambda i: (i, 0)),
  ]
  for c in range(C):
    in_specs.append(
        pl.BlockSpec((ROWS, 128),
                     functools.partial(lambda c, i: (c * blocks_per_chunk + i,
                                                     0), c)))
  in_specs.append(pl.BlockSpec((F, FH), lambda i: (0, 0)))

  return pl.pallas_call(
      body,
      grid=(GRID,),
      in_specs=in_specs,
      out_specs=[
          pl.BlockSpec((ROWS, FH), lambda i: (i, 0)),
          pl.BlockSpec((8, FH), lambda i: (0, 0)),
      ],
      out_shape=[
          jax.ShapeDtypeStruct((N, FH), jnp.float32),
          jax.ShapeDtypeStruct((8, FH), jnp.float32),
      ],
  )


@functools.cache
def _finish(FH, DOUT, do_tanh):
  def body(z_ref, stats_ref, g_ref, be_ref, w2_ref, b2_ref, h_ref):
    st = stats_ref[...]
    mu = st[0:1, :] / N
    var = jnp.maximum(st[1:2, :] / N - mu * mu, 0.0)
    zb = ((z_ref[...] - mu) * lax.rsqrt(var + BN_EPS) * g_ref[0:1, :]
          + be_ref[0:1, :])
    a = jnp.maximum(zb, 0.0)
    ho = jnp.dot(a, w2_ref[...],
                 preferred_element_type=jnp.float32) + b2_ref[0:1, :]
    if do_tanh:
      ho = jnp.tanh(ho)
    h_ref[...] = ho

  return pl.pallas_call(
      body,
      grid=(GRID,),
      in_specs=[
          pl.BlockSpec((ROWS, FH), lambda i: (i, 0)),
          pl.BlockSpec((8, FH), lambda i: (0, 0)),
          pl.BlockSpec((8, FH), lambda i: (0, 0)),
          pl.BlockSpec((8, FH), lambda i: (0, 0)),
          pl.BlockSpec((FH, DOUT), lambda i: (0, 0)),
          pl.BlockSpec((8, DOUT), lambda i: (0, 0)),
      ],
      out_specs=pl.BlockSpec((ROWS, DOUT), lambda i: (i, 0)),
      out_shape=jax.ShapeDtypeStruct((N, DOUT), jnp.float32),
  )


@functools.cache
def _gate(F):
  def body(bg2_ref, h_ref, wg1_ref, bg1_ref, wg2_ref, mask_ref, gate_ref,
           gmax_ref):
    i = pl.program_id(0)
    a = jnp.maximum(
        jnp.dot(h_ref[...], wg1_ref[...], preferred_element_type=jnp.float32)
        + bg1_ref[0:1, :], 0.0)
    g = jnp.dot(a, wg2_ref[...],
                preferred_element_type=jnp.float32) + bg2_ref[0]
    gate_ref[...] = g
    g0 = g[:, 0:1]
    contrib = jnp.where(mask_ref[...] > 0.0, g0, NEG)
    cm = jnp.max(contrib, axis=0, keepdims=True)
    upd = jnp.concatenate([cm, jnp.full((7, 128), NEG, jnp.float32)], axis=0)

    @pl.when(i == 0)
    def _():
      gmax_ref[...] = upd

    @pl.when(i > 0)
    def _():
      gmax_ref[...] = jnp.maximum(gmax_ref[...], upd)

  return pl.pallas_call(
      body,
      grid=(GRID,),
      in_specs=[
          pl.BlockSpec(memory_space=pltpu.SMEM),
          pl.BlockSpec((ROWS, F), lambda i: (i, 0)),
          pl.BlockSpec((F, F), lambda i: (0, 0)),
          pl.BlockSpec((8, F), lambda i: (0, 0)),
          pl.BlockSpec((F, 128), lambda i: (0, 0)),
          pl.BlockSpec((ROWS, 128), lambda i: (i, 0)),
      ],
      out_specs=[
          pl.BlockSpec((ROWS, 128), lambda i: (i, 0)),
          pl.BlockSpec((8, 128), lambda i: (0, 0)),
      ],
      out_shape=[
          jax.ShapeDtypeStruct((N, 128), jnp.float32),
          jax.ShapeDtypeStruct((8, 128), jnp.float32),
      ],
  )


@functools.cache
def _pool(F):
  W = F + 128

  def body(h_ref, gate_ref, mask_ref, gmax_ref, acc_ref):
    i = pl.program_id(0)
    gm = gmax_ref[0:1, :]
    gm = jnp.where(gm < NEG / 2, 0.0, gm)
    mask = mask_ref[...]
    gsel = jnp.sum(mask * gm, axis=1, keepdims=True)
    e = jnp.exp(gate_ref[:, 0:1] - gsel)
    eh = jnp.concatenate(
        [e * h_ref[...], e,
         jnp.zeros((ROWS, 127), jnp.float32)], axis=1)
    upd = lax.dot_general(mask, eh, (((0,), (0,)), ((), ())),
                          preferred_element_type=jnp.float32)

    @pl.when(i == 0)
    def _():
      acc_ref[...] = upd

    @pl.when(i > 0)
    def _():
      acc_ref[...] = acc_ref[...] + upd

  return pl.pallas_call(
      body,
      grid=(GRID,),
      in_specs=[
          pl.BlockSpec((ROWS, F), lambda i: (i, 0)),
          pl.BlockSpec((ROWS, 128), lambda i: (i, 0)),
          pl.BlockSpec((ROWS, 128), lambda i: (i, 0)),
          pl.BlockSpec((8, 128), lambda i: (0, 0)),
      ],
      out_specs=pl.BlockSpec((128, W), lambda i: (0, 0)),
      out_shape=jax.ShapeDtypeStruct((128, W), jnp.float32),
  )


@functools.cache
def _head(F):
  W = F + 128

  def _bn(g, gamma, beta):
    mu = jnp.mean(g, axis=0, keepdims=True)
    xc = g - mu
    var = jnp.mean(xc * xc, axis=0, keepdims=True)
    return xc * lax.rsqrt(var + BN_EPS) * gamma + beta

  def body(bout_ref, acc_ref, w0, b0, w1, b1, w2, b2, g0, e0, g1, e1, g2, e2,
           g3, e3, wout_ref, out_ref):
    acc = acc_ref[...]
    g = acc[0:NG, 0:F] / (acc[0:NG, F:F + 1] + 1e-16)
    lins = ((w0, b0, g0, e0), (w1, b1, g1, e1), (w2, b2, g2, e2))
    for w, b, ga, be in lins:
      g = _bn(g, ga[0:1, :], be[0:1, :])
      g = jnp.dot(g, w[...], preferred_element_type=jnp.float32) + b[0:1, :]
      g = jnp.tanh(g)
    g = _bn(g, g3[0:1, :], e3[0:1, :])
    out = jnp.dot(g, wout_ref[...],
                  preferred_element_type=jnp.float32) + bout_ref[0]
    out_ref[...] = out

  full = lambda shape: pl.BlockSpec(shape, lambda: (0, 0))
  in_specs = [pl.BlockSpec(memory_space=pltpu.SMEM), full((128, W))]
  for _ in range(3):
    in_specs += [full((F, F)), full((8, F))]
  for _ in range(4):
    in_specs += [full((8, F)), full((8, F))]
  in_specs.append(full((F, 128)))

  return pl.pallas_call(
      body,
      grid=(),
      in_specs=in_specs,
      out_specs=full((NG, 128)),
      out_shape=jax.ShapeDtypeStruct((NG, 128), jnp.float32),
  )


def _bcast8(v):
  return jnp.broadcast_to(v.reshape(1, -1), (8, v.shape[-1]))


def kernel(x, edge_index, batch, params):
  src = edge_index[0]
  dst = edge_index[1]

  srcp = jnp.concatenate(
      [src.reshape(SUB, EPT),
       jnp.zeros((SUB, EPT_PAD - EPT), jnp.int32)], axis=1)
  dstp = jnp.concatenate(
      [dst.reshape(SUB, EPT),
       jnp.broadcast_to(N + jnp.arange(SUB, dtype=jnp.int32)[:, None],
                        (SUB, EPT_PAD - EPT))], axis=1)
  src_c = (jnp.arange(8, dtype=jnp.int32)[:, None, None] * N +
           srcp[None]).reshape(8, SUB, NBLK, BLK)
  dst_c = jnp.broadcast_to(dstp[None],
                           (8, SUB, EPT_PAD)).reshape(8, SUB, NBLK, BLK)
  pack8 = jnp.stack([src_c, dst_c], axis=3).reshape(8 * SUB, NBLK, 2, BLK)
  zeros_pad = jnp.zeros((ACC // SUB, 128), jnp.float32)
  bmask = (batch[:, None] == jnp.arange(NG, dtype=jnp.int32)[None, :])
  bmask = jnp.concatenate(
      [bmask.astype(jnp.float32),
       jnp.zeros((N, 128 - NG), jnp.float32)], axis=1)

  h = x
  for li, p in enumerate(params['convs']):
    F = h.shape[1]
    FH = 2 * F
    C = 2 * F // 128
    table = _prep(F)(p['t'].reshape(1), h).reshape(C * N, 128)
    scout = _sc_edge(C)(table, pack8[:C * SUB], zeros_pad)
    z, stats = _combine(F, FH)(p['s'].reshape(1), h, *([scout] * C), p['W1'])
    h = _finish(FH, p['W2'].shape[1], li < 2)(z, stats, _bcast8(p['g1']),
                                              _bcast8(p['be1']), p['W2'],
                                              _bcast8(p['b2']))

  F = h.shape[1]
  wg2 = jnp.pad(params['Wg2'], ((0, 0), (0, 127)))
  gate, gmax = _gate(F)(params['bg2'], h, params['Wg1'],
                        _bcast8(params['bg1']), wg2, bmask)
  acc = _pool(F)(h, gate, bmask, gmax)

  wout = jnp.pad(params['Wout'], ((0, 0), (0, 127)))
  head_args = [params['bout'], acc]
  for lin in params['lins']:
    head_args += [lin['W'], _bcast8(lin['b'])]
  for bn in params['bns']:
    head_args += [_bcast8(bn['g']), _bcast8(bn['be'])]
  head_args.append(wout)
  out = _head(F)(*head_args)
  return out[:, 0:1]
